# Initial kernel scaffold; baseline (speedup 1.0000x reference)
import jax
import jax.numpy as jnp
from jax.experimental import pallas as pl


def kernel(token_ids, weight):
    raise NotImplementedError("write your pallas kernel here")


# speedup vs baseline: 1.0944x; 1.0944x over previous
import functools

import jax
import jax.numpy as jnp
from jax import lax
from jax.experimental import pallas as pl
from jax.experimental.pallas import tpu as pltpu
from jax.experimental.pallas import tpu_sc as plsc

B, S = 16384, 50
V, D = 1_000_000, 32
N = B * S
IDXW = 128
NROWS = N // IDXW

NC, NS = 2, 16
NW = NC * NS
ROWS_PER_W = NROWS // NW
G = 8
CHUNK = G * IDXW
NCHUNKS = ROWS_PER_W // G


def _make_sc_lookup():
  mesh = plsc.VectorSubcoreMesh(core_axis_name="c", subcore_axis_name="s")

  @functools.partial(
      pl.kernel,
      mesh=mesh,
      compiler_params=pltpu.CompilerParams(use_tc_tiling_on_sc=False),
      out_type=jax.ShapeDtypeStruct((N, D), jnp.float32),
      scratch_types=[
          pltpu.VMEM((G, IDXW), jnp.int32),
          pltpu.VMEM((CHUNK, D), jnp.float32),
          pltpu.SemaphoreType.DMA,
      ],
  )
  def lookup(idx_hbm, table_hbm, out_hbm, idx_v, rows_v, sem):
    wid = lax.axis_index("s") * NC + lax.axis_index("c")
    row_base = wid * ROWS_PER_W

    def chunk_body(c, carry):
      r0 = row_base + c * G
      pltpu.sync_copy(idx_hbm.at[pl.ds(r0, G)], idx_v)
      copies = []
      for j in range(G):
        copies.append(
            pltpu.async_copy(
                table_hbm.at[idx_v.at[j]],
                rows_v.at[pl.ds(j * IDXW, IDXW)],
                sem,
            ))
      for cp in copies:
        cp.wait()
      pltpu.sync_copy(rows_v, out_hbm.at[pl.ds(r0 * IDXW, CHUNK)])
      return carry

    lax.fori_loop(0, NCHUNKS, chunk_body, 0)

  return lookup


_sc_lookup = _make_sc_lookup()


@jax.jit
def kernel(token_ids, weight):
  idx = token_ids.astype(jnp.int32).reshape(NROWS, IDXW)
  out = _sc_lookup(idx, weight)
  return out.reshape(B, S, D)


# speedup vs baseline: 1.1108x; 1.0150x over previous
import functools

import jax
import jax.numpy as jnp
from jax import lax
from jax.experimental import pallas as pl
from jax.experimental.pallas import tpu as pltpu
from jax.experimental.pallas import tpu_sc as plsc

B, S = 16384, 50
V, D = 1_000_000, 32
N = B * S
IDXW = 128
NROWS = N // IDXW

NC, NS = 2, 16
NW = NC * NS
ROWS_PER_W = NROWS // NW
G = 10
CHUNK = G * IDXW
NCHUNKS = ROWS_PER_W // G


def _make_sc_lookup():
  mesh = plsc.VectorSubcoreMesh(core_axis_name="c", subcore_axis_name="s")

  @functools.partial(
      pl.kernel,
      mesh=mesh,
      compiler_params=pltpu.CompilerParams(use_tc_tiling_on_sc=False),
      out_type=jax.ShapeDtypeStruct((N, D), jnp.float32),
      scratch_types=[
          pltpu.VMEM((G, IDXW), jnp.int32),
          pltpu.VMEM((G, IDXW), jnp.int32),
          pltpu.VMEM((CHUNK, D), jnp.float32),
          pltpu.VMEM((CHUNK, D), jnp.float32),
          pltpu.SemaphoreType.DMA,
          pltpu.SemaphoreType.DMA,
          pltpu.SemaphoreType.DMA,
          pltpu.SemaphoreType.DMA,
          pltpu.SemaphoreType.DMA,
          pltpu.SemaphoreType.DMA,
      ],
  )
  def lookup(idx_hbm, table_hbm, out_hbm, idx0, idx1, rows0, rows1,
             isem0, isem1, gsem0, gsem1, osem0, osem1):
    wid = lax.axis_index("s") * NC + lax.axis_index("c")
    row_base = wid * ROWS_PER_W
    idx_v = (idx0, idx1)
    rows_v = (rows0, rows1)
    isem = (isem0, isem1)
    gsem = (gsem0, gsem1)
    osem = (osem0, osem1)

    def idx_copy(c, s):
      return pltpu.make_async_copy(
          idx_hbm.at[pl.ds(row_base + c * G, G)], idx_v[s], isem[s])

    def out_copy(c, s):
      return pltpu.make_async_copy(
          rows_v[s], out_hbm.at[pl.ds((row_base + c * G) * IDXW, CHUNK)],
          osem[s])

    idx_copy(0, 0).start()
    idx_copy(1, 1).start()

    def pair_body(i, carry):
      for s in range(2):
        c = 2 * i + s
        idx_copy(c, s).wait()
        @pl.when(c >= 2)
        def _():
          out_copy(c - 2, s).wait()
        gathers = []
        for j in range(G):
          gathers.append(
              pltpu.async_copy(
                  table_hbm.at[idx_v[s].at[j]],
                  rows_v[s].at[pl.ds(j * IDXW, IDXW)],
                  gsem[s],
              ))
        for cp in gathers:
          cp.wait()
        @pl.when(c + 2 < NCHUNKS)
        def _():
          idx_copy(c + 2, s).start()
        out_copy(c, s).start()
      return carry

    lax.fori_loop(0, NCHUNKS // 2, pair_body, 0)
    out_copy(NCHUNKS - 2, 0).wait()
    out_copy(NCHUNKS - 1, 1).wait()

  return lookup


_sc_lookup = _make_sc_lookup()


@jax.jit
def kernel(token_ids, weight):
  idx = token_ids.astype(jnp.int32).reshape(NROWS, IDXW)
  out = _sc_lookup(idx, weight)
  return out.reshape(B, S, D)


# speedup vs baseline: 1.7982x; 1.6187x over previous
import functools

import jax
import jax.numpy as jnp
from jax import lax
from jax.experimental import pallas as pl
from jax.experimental.pallas import tpu as pltpu
from jax.experimental.pallas import tpu_sc as plsc

B, S = 16384, 50
V, D = 1_000_000, 32

NC, NS = 2, 16
NW = NC * NS
ROWS_PER_W = B // NW
R = 16
NCHUNKS = ROWS_PER_W // R


def _make_sc_lookup():
  mesh = plsc.VectorSubcoreMesh(core_axis_name="c", subcore_axis_name="s")

  @functools.partial(
      pl.kernel,
      mesh=mesh,
      compiler_params=pltpu.CompilerParams(use_tc_tiling_on_sc=False),
      out_type=jax.ShapeDtypeStruct((B, S, D), jnp.float32),
      scratch_types=[
          pltpu.VMEM((R, S), jnp.int32),
          pltpu.VMEM((R, S), jnp.int32),
          pltpu.VMEM((R, S, D), jnp.float32),
          pltpu.VMEM((R, S, D), jnp.float32),
          pltpu.SemaphoreType.DMA,
          pltpu.SemaphoreType.DMA,
          pltpu.SemaphoreType.DMA,
          pltpu.SemaphoreType.DMA,
          pltpu.SemaphoreType.DMA,
          pltpu.SemaphoreType.DMA,
      ],
  )
  def lookup(idx_hbm, table_hbm, out_hbm, idx0, idx1, rows0, rows1,
             isem0, isem1, gsem0, gsem1, osem0, osem1):
    wid = lax.axis_index("s") * NC + lax.axis_index("c")
    row_base = wid * ROWS_PER_W
    idx_v = (idx0, idx1)
    rows_v = (rows0, rows1)
    isem = (isem0, isem1)
    gsem = (gsem0, gsem1)
    osem = (osem0, osem1)

    def idx_copy(c, s):
      return pltpu.make_async_copy(
          idx_hbm.at[pl.ds(row_base + c * R, R)], idx_v[s], isem[s])

    def out_copy(c, s):
      return pltpu.make_async_copy(
          rows_v[s], out_hbm.at[pl.ds(row_base + c * R, R)], osem[s])

    idx_copy(0, 0).start()
    idx_copy(1, 1).start()

    def pair_body(i, carry):
      for s in range(2):
        c = 2 * i + s
        idx_copy(c, s).wait()
        @pl.when(c >= 2)
        def _():
          out_copy(c - 2, s).wait()
        gathers = []
        for j in range(R):
          gathers.append(
              pltpu.async_copy(
                  table_hbm.at[idx_v[s].at[j]],
                  rows_v[s].at[j],
                  gsem[s],
              ))
        for cp in gathers:
          cp.wait()
        @pl.when(c + 2 < NCHUNKS)
        def _():
          idx_copy(c + 2, s).start()
        out_copy(c, s).start()
      return carry

    lax.fori_loop(0, NCHUNKS // 2, pair_body, 0)
    out_copy(NCHUNKS - 2, 0).wait()
    out_copy(NCHUNKS - 1, 1).wait()

  return lookup


_sc_lookup = _make_sc_lookup()


@jax.jit
def kernel(token_ids, weight):
  return _sc_lookup(token_ids.astype(jnp.int32), weight)


# speedup vs baseline: 1.8012x; 1.0017x over previous
import functools

import jax
import jax.numpy as jnp
from jax import lax
from jax.experimental import pallas as pl
from jax.experimental.pallas import tpu as pltpu
from jax.experimental.pallas import tpu_sc as plsc

B, S = 16384, 50
V, D = 1_000_000, 32

NC, NS = 2, 16
NW = NC * NS
ROWS_PER_W = B // NW
CB = 32
NCHUNKS = ROWS_PER_W // CB
BT = 128
NBT = B // BT
DBLK = D // 8


def _make_sc_lookup():
  mesh = plsc.VectorSubcoreMesh(core_axis_name="c", subcore_axis_name="s")

  @functools.partial(
      pl.kernel,
      mesh=mesh,
      compiler_params=pltpu.CompilerParams(
          use_tc_tiling_on_sc=False, needs_layout_passes=False),
      out_type=jax.ShapeDtypeStruct((S, DBLK, NBT, 8, BT), jnp.float32),
      scratch_types=[
          pltpu.VMEM((CB, S), jnp.int32),
          pltpu.VMEM((CB, S), jnp.int32),
          pltpu.VMEM((CB, S, D), jnp.float32),
          pltpu.VMEM((CB, S, D), jnp.float32),
          pltpu.VMEM((D, CB), jnp.float32),
          pltpu.VMEM((D, CB), jnp.float32),
          pltpu.SemaphoreType.DMA,
          pltpu.SemaphoreType.DMA,
          pltpu.SemaphoreType.DMA,
          pltpu.SemaphoreType.DMA,
          pltpu.SemaphoreType.DMA,
          pltpu.SemaphoreType.DMA,
      ],
  )
  def lookup(idx_hbm, table_hbm, out_hbm, idx0, idx1, rows0, rows1,
             st0, st1, isem0, isem1, gsem0, gsem1, ssem0, ssem1):
    wid = lax.axis_index("s") * NC + lax.axis_index("c")
    b_base = wid * ROWS_PER_W
    idx_v = (idx0, idx1)
    rows_v = (rows0, rows1)
    stage = (st0, st1)
    isem = (isem0, isem1)
    gsem = (gsem0, gsem1)
    ssem = (ssem0, ssem1)

    lane = lax.iota(jnp.int32, 16)
    zero16 = jnp.zeros((16,), jnp.int32)
    d_c = [lane, lane + 16]

    def idx_copy(c, s):
      return pltpu.make_async_copy(
          idx_hbm.at[pl.ds(b_base + c * CB, CB)], idx_v[s], isem[s])

    def fire_gathers(s):
      def go(j, carry):
        pltpu.make_async_copy(
            table_hbm.at[idx_v[s].at[j]], rows_v[s].at[j], gsem[s]).start()
        return carry
      lax.fori_loop(0, CB, go, 0)

    def drain_gathers(s):
      def dr(j, carry):
        pltpu.make_async_copy(
            table_hbm.at[idx_v[s].at[j]], rows_v[s].at[j], gsem[s]).wait()
        return carry
      lax.fori_loop(0, CB, dr, 0)

    def stage_copies(c, s_, q):
      btile = (b_base + c * CB) // BT
      bcol0 = (b_base + c * CB) % BT
      return [
          pltpu.make_async_copy(
              stage[q].at[pl.ds(db * 8, 8)],
              out_hbm.at[s_, db, btile, pl.ds(0, 8), pl.ds(bcol0, CB)],
              ssem[q])
          for db in range(DBLK)
      ]

    def transpose_chunk(c, s):
      def per_pair(p, carry):
        for q in range(2):
          s_ = 2 * p + q
          @pl.when(s_ >= 2)
          def _():
            for cp in stage_copies(c, s_ - 2, q):
              cp.wait()
          for bb in range(CB):
            bb_c = zero16 + bb
            for h in range(2):
              vals = rows_v[s][bb, s_, pl.ds(16 * h, 16)]
              plsc.store_scatter(stage[q], [d_c[h], bb_c], vals)
          for cp in stage_copies(c, s_, q):
            cp.start()
        return carry
      lax.fori_loop(0, S // 2, per_pair, 0)
      for cp in stage_copies(c, S - 2, 0):
        cp.wait()
      for cp in stage_copies(c, S - 1, 1):
        cp.wait()

    idx_copy(0, 0).start()
    idx_copy(1, 1).start()
    idx_copy(0, 0).wait()
    fire_gathers(0)

    def pair_body(i, carry):
      for s in range(2):
        c = 2 * i + s
        @pl.when(c + 1 < NCHUNKS)
        def _():
          idx_copy(c + 1, 1 - s).wait()
          fire_gathers(1 - s)
        drain_gathers(s)
        @pl.when(c + 2 < NCHUNKS)
        def _():
          idx_copy(c + 2, s).start()
        transpose_chunk(c, s)
      return carry

    lax.fori_loop(0, NCHUNKS // 2, pair_body, 0)

  return lookup


_sc_lookup = _make_sc_lookup()


@jax.jit
def kernel(token_ids, weight):
  out5 = _sc_lookup(token_ids.astype(jnp.int32), weight)
  return out5.transpose(2, 4, 0, 1, 3).reshape(B, S, D)


# speedup vs baseline: 1.8115x; 1.0057x over previous
import functools

import jax
import jax.numpy as jnp
from jax import lax
from jax.experimental import pallas as pl
from jax.experimental.pallas import tpu as pltpu
from jax.experimental.pallas import tpu_sc as plsc

B, S = 16384, 50
V, D = 1_000_000, 32

NC, NS = 2, 16
NW = NC * NS
ROWS_PER_W = B // NW
CB = 32
NCHUNKS = ROWS_PER_W // CB
BT = 128
NBT = B // BT
DBLK = D // 8


def _make_sc_lookup():
  mesh = plsc.VectorSubcoreMesh(core_axis_name="c", subcore_axis_name="s")

  @functools.partial(
      pl.kernel,
      mesh=mesh,
      compiler_params=pltpu.CompilerParams(
          use_tc_tiling_on_sc=False, needs_layout_passes=False),
      out_type=jax.ShapeDtypeStruct((S, DBLK, NBT, 8, BT), jnp.float32),
      scratch_types=[
          pltpu.VMEM((CB, S), jnp.int32),
          pltpu.VMEM((CB, S), jnp.int32),
          pltpu.VMEM((CB, S, D), jnp.float32),
          pltpu.VMEM((CB, S, D), jnp.float32),
          pltpu.VMEM((DBLK, 1, 8, CB), jnp.float32),
          pltpu.VMEM((DBLK, 1, 8, CB), jnp.float32),
          pltpu.SemaphoreType.DMA,
          pltpu.SemaphoreType.DMA,
          pltpu.SemaphoreType.DMA,
          pltpu.SemaphoreType.DMA,
          pltpu.SemaphoreType.DMA,
          pltpu.SemaphoreType.DMA,
      ],
  )
  def lookup(idx_hbm, table_hbm, out_hbm, idx0, idx1, rows0, rows1,
             st0, st1, isem0, isem1, gsem0, gsem1, ssem0, ssem1):
    wid = lax.axis_index("s") * NC + lax.axis_index("c")
    b_base = wid * ROWS_PER_W
    idx_v = (idx0, idx1)
    rows_v = (rows0, rows1)
    stage = (st0, st1)
    isem = (isem0, isem1)
    gsem = (gsem0, gsem1)
    ssem = (ssem0, ssem1)

    lane = lax.iota(jnp.int32, 16)
    zero16 = jnp.zeros((16,), jnp.int32)
    drow_lo = lane
    drow_hi = lane - 8
    m_lo = lane < 8
    m_hi = lane >= 8

    def idx_copy(c, s):
      return pltpu.make_async_copy(
          idx_hbm.at[pl.ds(b_base + c * CB, CB)], idx_v[s], isem[s])

    def fire_gathers(s):
      def go(j, carry):
        pltpu.make_async_copy(
            table_hbm.at[idx_v[s].at[j]], rows_v[s].at[j], gsem[s]).start()
        return carry
      lax.fori_loop(0, CB, go, 0)

    def drain_gathers(s):
      def dr(j, carry):
        pltpu.make_async_copy(
            table_hbm.at[idx_v[s].at[j]], rows_v[s].at[j], gsem[s]).wait()
        return carry
      lax.fori_loop(0, CB, dr, 0)

    def stage_copies(c, s_, q):
      btile = (b_base + c * CB) // BT
      bcol0 = (b_base + c * CB) % BT
      return [
          pltpu.make_async_copy(
              stage[q],
              out_hbm.at[s_, pl.ds(0, DBLK), pl.ds(btile, 1), pl.ds(0, 8),
                         pl.ds(bcol0, CB)],
              ssem[q])
      ]

    def transpose_chunk(c, s):
      def per_pair(p, carry):
        for q in range(2):
          s_ = 2 * p + q
          @pl.when(s_ >= 2)
          def _():
            for cp in stage_copies(c, s_ - 2, q):
              cp.wait()
          for bb in range(CB):
            bb_c = zero16 + bb
            for h in range(2):
              vals = rows_v[s][bb, s_, pl.ds(16 * h, 16)]
              plsc.store_scatter(stage[q].at[2 * h, 0],
                                 [drow_lo, bb_c], vals, mask=m_lo)
              plsc.store_scatter(stage[q].at[2 * h + 1, 0],
                                 [drow_hi, bb_c], vals, mask=m_hi)
          for cp in stage_copies(c, s_, q):
            cp.start()
        return carry
      lax.fori_loop(0, S // 2, per_pair, 0)
      for cp in stage_copies(c, S - 2, 0):
        cp.wait()
      for cp in stage_copies(c, S - 1, 1):
        cp.wait()

    idx_copy(0, 0).start()
    idx_copy(1, 1).start()
    idx_copy(0, 0).wait()
    fire_gathers(0)

    def pair_body(i, carry):
      for s in range(2):
        c = 2 * i + s
        @pl.when(c + 1 < NCHUNKS)
        def _():
          idx_copy(c + 1, 1 - s).wait()
          fire_gathers(1 - s)
        drain_gathers(s)
        @pl.when(c + 2 < NCHUNKS)
        def _():
          idx_copy(c + 2, s).start()
        transpose_chunk(c, s)
      return carry

    lax.fori_loop(0, NCHUNKS // 2, pair_body, 0)

  return lookup


_sc_lookup = _make_sc_lookup()


@jax.jit
def kernel(token_ids, weight):
  out5 = _sc_lookup(token_ids.astype(jnp.int32), weight)
  return out5.transpose(2, 4, 0, 1, 3).reshape(B, S, D)


# speedup vs baseline: 1.8133x; 1.0010x over previous
import functools

import jax
import jax.numpy as jnp
from jax import lax
from jax.experimental import pallas as pl
from jax.experimental.pallas import tpu as pltpu
from jax.experimental.pallas import tpu_sc as plsc

B, S = 16384, 50
V, D = 1_000_000, 32

NC, NS = 2, 16
NW = NC * NS
ROWS_PER_W = B // NW
CB = 32
NCHUNKS = ROWS_PER_W // CB
BT = 128
NBT = B // BT
DBLK = D // 8


def _make_sc_lookup():
  mesh = plsc.VectorSubcoreMesh(core_axis_name="c", subcore_axis_name="s")

  @functools.partial(
      pl.kernel,
      mesh=mesh,
      compiler_params=pltpu.CompilerParams(
          use_tc_tiling_on_sc=False, needs_layout_passes=False),
      out_type=jax.ShapeDtypeStruct((S, DBLK, NBT, 8, BT), jnp.float32),
      scratch_types=[
          pltpu.VMEM((CB, S), jnp.int32),
          pltpu.VMEM((CB, S), jnp.int32),
          pltpu.VMEM((CB, S, D), jnp.float32),
          pltpu.VMEM((CB, S, D), jnp.float32),
          pltpu.VMEM((DBLK, 1, 8, CB), jnp.float32),
          pltpu.VMEM((DBLK, 1, 8, CB), jnp.float32),
          pltpu.SemaphoreType.DMA,
          pltpu.SemaphoreType.DMA,
          pltpu.SemaphoreType.DMA,
          pltpu.SemaphoreType.DMA,
          pltpu.SemaphoreType.DMA,
          pltpu.SemaphoreType.DMA,
      ],
  )
  def lookup(idx_hbm, table_hbm, out_hbm, idx0, idx1, rows0, rows1,
             st0, st1, isem0, isem1, gsem0, gsem1, ssem0, ssem1):
    wid = lax.axis_index("s") * NC + lax.axis_index("c")
    b_base = wid * ROWS_PER_W
    idx_v = (idx0, idx1)
    rows_v = (rows0, rows1)
    stage = (st0, st1)
    isem = (isem0, isem1)
    gsem = (gsem0, gsem1)
    ssem = (ssem0, ssem1)

    lane = lax.iota(jnp.int32, 16)
    zero16 = jnp.zeros((16,), jnp.int32)
    dblk_c = [lane // 8, lane // 8 + 2]
    drow_c = lax.rem(lane, 8)

    def idx_copy(c, s):
      return pltpu.make_async_copy(
          idx_hbm.at[pl.ds(b_base + c * CB, CB)], idx_v[s], isem[s])

    def fire_gathers(s):
      def go(j, carry):
        pltpu.make_async_copy(
            table_hbm.at[idx_v[s].at[j]], rows_v[s].at[j], gsem[s]).start()
        return carry
      lax.fori_loop(0, CB, go, 0)

    def drain_gathers(s):
      def dr(j, carry):
        pltpu.make_async_copy(
            table_hbm.at[idx_v[s].at[j]], rows_v[s].at[j], gsem[s]).wait()
        return carry
      lax.fori_loop(0, CB, dr, 0)

    def stage_copies(c, s_, q):
      btile = (b_base + c * CB) // BT
      bcol0 = (b_base + c * CB) % BT
      return [
          pltpu.make_async_copy(
              stage[q],
              out_hbm.at[s_, pl.ds(0, DBLK), pl.ds(btile, 1), pl.ds(0, 8),
                         pl.ds(bcol0, CB)],
              ssem[q])
      ]

    def transpose_chunk(c, s):
      def per_pair(p, carry):
        for q in range(2):
          s_ = 2 * p + q
          @pl.when(s_ >= 2)
          def _():
            for cp in stage_copies(c, s_ - 2, q):
              cp.wait()
          for bb in range(CB):
            bb_c = zero16 + bb
            for h in range(2):
              vals = rows_v[s][bb, s_, pl.ds(16 * h, 16)]
              plsc.store_scatter(
                  stage[q], [dblk_c[h], zero16, drow_c, bb_c], vals)
          for cp in stage_copies(c, s_, q):
            cp.start()
        return carry
      lax.fori_loop(0, S // 2, per_pair, 0)
      for cp in stage_copies(c, S - 2, 0):
        cp.wait()
      for cp in stage_copies(c, S - 1, 1):
        cp.wait()

    idx_copy(0, 0).start()
    idx_copy(1, 1).start()
    idx_copy(0, 0).wait()
    fire_gathers(0)

    def pair_body(i, carry):
      for s in range(2):
        c = 2 * i + s
        @pl.when(c + 1 < NCHUNKS)
        def _():
          idx_copy(c + 1, 1 - s).wait()
          fire_gathers(1 - s)
        drain_gathers(s)
        @pl.when(c + 2 < NCHUNKS)
        def _():
          idx_copy(c + 2, s).start()
        transpose_chunk(c, s)
      return carry

    lax.fori_loop(0, NCHUNKS // 2, pair_body, 0)

  return lookup


_sc_lookup = _make_sc_lookup()


@jax.jit
def kernel(token_ids, weight):
  out5 = _sc_lookup(token_ids.astype(jnp.int32), weight)
  return out5.transpose(2, 4, 0, 1, 3).reshape(B, S, D)
